# baseline (speedup 1.0000x reference)
import functools

import jax
import jax.numpy as jnp
from jax import lax
from jax.experimental import pallas as pl
from jax.experimental.pallas import tpu as pltpu
from jax.experimental.pallas import tpu_sc as plsc

B, P, T = 4096, 512, 50
J = 64
NW = 32
B_PER_W = B // NW
ELEMS_PER_W = B_PER_W * J


def _sc_gather(latent_t, positions):
    mesh = plsc.VectorSubcoreMesh(core_axis_name="c", subcore_axis_name="s")

    @functools.partial(
        pl.kernel,
        out_type=jax.ShapeDtypeStruct((B * J,), jnp.float32),
        mesh=mesh,
        scratch_types=[
            pltpu.VMEM((J,), jnp.int32),
            pltpu.VMEM((B_PER_W, P), jnp.float32),
            pltpu.VMEM((ELEMS_PER_W,), jnp.float32),
            pltpu.SemaphoreType.DMA,
        ],
        compiler_params=pltpu.CompilerParams(needs_layout_passes=False),
    )
    def k(lat_hbm, pos_hbm, out_hbm, pos_v, buf_v, outst_v, sem):
        wid = lax.axis_index("s") * 2 + lax.axis_index("c")
        b0 = wid * B_PER_W

        pltpu.sync_copy(pos_hbm, pos_v)
        pltpu.sync_copy(lat_hbm.at[0, pl.ds(b0, B_PER_W)], buf_v)

        p0 = pos_v[pl.ds(0, 16)]
        p1 = pos_v[pl.ds(16, 16)]
        p2 = pos_v[pl.ds(32, 16)]
        p3 = pos_v[pl.ds(48, 16)]

        def extract(bl, carry):
            q0, q1, q2, q3 = carry
            row = jnp.full((16,), bl, jnp.int32)
            outst_v[pl.ds(bl * J, 16)] = plsc.load_gather(buf_v, [row, q0])
            outst_v[pl.ds(bl * J + 16, 16)] = plsc.load_gather(buf_v, [row, q1])
            outst_v[pl.ds(bl * J + 32, 16)] = plsc.load_gather(buf_v, [row, q2])
            outst_v[pl.ds(bl * J + 48, 16)] = plsc.load_gather(buf_v, [row, q3])
            return carry

        lax.fori_loop(0, 1, extract, (p0, p1, p2, p3), unroll=1)

        pltpu.sync_copy(outst_v, out_hbm.at[pl.ds(wid * ELEMS_PER_W, ELEMS_PER_W)])

    return k(latent_t, positions)


@jax.jit
def kernel(latent_space, bit_positions):
    latent_t = jnp.transpose(latent_space, (2, 0, 1))
    positions = bit_positions.astype(jnp.int32)
    out = _sc_gather(latent_t, positions)
    return out.reshape(B, J)
